# baseline (speedup 1.0000x reference)
import jax
import jax.numpy as jnp
from jax import lax
from jax.experimental import pallas as pl
from jax.experimental.pallas import tpu as pltpu
from jax.experimental.pallas import tpu_sc as plsc

C, T, H, W = 3, 64, 256, 256
S = T // 4
FRAME = H * W
HALF = FRAME // 2
N_HALF = C * S * 2

_info = plsc.get_sparse_core_info()
NW = _info.num_cores * _info.num_subcores
PER_W = N_HALF // NW


def _tc_copy_body(in_ref, out_ref):
    out_ref[...] = in_ref[...]


def _fast_copy(frames):
    FB = 8
    return pl.pallas_call(
        _tc_copy_body,
        grid=(C, T // FB),
        in_specs=[pl.BlockSpec((1, FB, H, W), lambda c, b: (c, b, 0, 0))],
        out_specs=pl.BlockSpec((1, FB, H, W), lambda c, b: (c, b, 0, 0)),
        out_shape=jax.ShapeDtypeStruct((C, T, H, W), frames.dtype),
    )(frames)


HROWS = H // 2


def _sc_gather(frames_2d):
    mesh = plsc.VectorSubcoreMesh(core_axis_name="c", subcore_axis_name="s")

    @pl.kernel(
        out_type=jax.ShapeDtypeStruct((C * S * H, W), jnp.float32),
        mesh=mesh,
        scratch_types=[
            pltpu.VMEM((HROWS, W), jnp.float32),
            pltpu.VMEM((HROWS, W), jnp.float32),
            pltpu.VMEM((HROWS, W), jnp.float32),
            pltpu.SemaphoreType.DMA,
            pltpu.SemaphoreType.DMA,
        ],
    )
    def k(frames_hbm, slow_hbm, buf0, buf1, buf2, in_sem, out_sem):
        bufs = [buf0, buf1, buf2]
        wid = lax.axis_index("s") * _info.num_cores + lax.axis_index("c")
        ins, outs = [], []
        for i in range(PER_W):
            h = wid * PER_W + i
            s = h // 2
            half = h % 2
            c = s // S
            j = s % S
            t = (21 * j) // 5
            src_row = ((c * T + t) * 2 + half) * HROWS
            dst_row = h * HROWS
            ins.append(
                pltpu.make_async_copy(
                    frames_hbm.at[pl.ds(src_row, HROWS)], bufs[i], in_sem
                )
            )
            outs.append(
                pltpu.make_async_copy(
                    bufs[i], slow_hbm.at[pl.ds(dst_row, HROWS)], out_sem
                )
            )
        for cp in ins:
            cp.start()
        for i in range(PER_W):
            ins[i].wait()
            outs[i].start()
        for cp in outs:
            cp.wait()

    return k(frames_2d)


def kernel(frames):
    fast = _fast_copy(frames)
    slow = _sc_gather(frames.reshape(C * T * H, W)).reshape(C, S, H, W)
    return (slow, fast)


# speedup vs baseline: 1.0620x; 1.0620x over previous
import jax
import jax.numpy as jnp
from jax import lax
from jax.experimental import pallas as pl
from jax.experimental.pallas import tpu as pltpu
from jax.experimental.pallas import tpu_sc as plsc

C, T, H, W = 3, 64, 256, 256
S = T // 4
FRAME = H * W
HALF = FRAME // 2
N_HALF = C * S * 2

_info = plsc.get_sparse_core_info()
NW = _info.num_cores * _info.num_subcores
PER_W = N_HALF // NW


def _tc_copy_body(in_ref, out_ref):
    out_ref[...] = in_ref[...]


def _fast_copy(frames):
    FB = 32
    return pl.pallas_call(
        _tc_copy_body,
        grid=(C, T // FB),
        in_specs=[pl.BlockSpec((1, FB, H, W), lambda c, b: (c, b, 0, 0))],
        out_specs=pl.BlockSpec((1, FB, H, W), lambda c, b: (c, b, 0, 0)),
        out_shape=jax.ShapeDtypeStruct((C, T, H, W), frames.dtype),
    )(frames)


HROWS = H // 2


def _sc_gather(frames_2d):
    mesh = plsc.VectorSubcoreMesh(core_axis_name="c", subcore_axis_name="s")

    @pl.kernel(
        out_type=jax.ShapeDtypeStruct((C * S * H, W), jnp.float32),
        mesh=mesh,
        scratch_types=[
            pltpu.VMEM((HROWS, W), jnp.float32),
            pltpu.VMEM((HROWS, W), jnp.float32),
            pltpu.VMEM((HROWS, W), jnp.float32),
            pltpu.SemaphoreType.DMA,
            pltpu.SemaphoreType.DMA,
        ],
    )
    def k(frames_hbm, slow_hbm, buf0, buf1, buf2, in_sem, out_sem):
        bufs = [buf0, buf1, buf2]
        wid = lax.axis_index("s") * _info.num_cores + lax.axis_index("c")
        ins, outs = [], []
        for i in range(PER_W):
            h = wid * PER_W + i
            s = h // 2
            half = h % 2
            c = s // S
            j = s % S
            t = (21 * j) // 5
            src_row = ((c * T + t) * 2 + half) * HROWS
            dst_row = h * HROWS
            ins.append(
                pltpu.make_async_copy(
                    frames_hbm.at[pl.ds(src_row, HROWS)], bufs[i], in_sem
                )
            )
            outs.append(
                pltpu.make_async_copy(
                    bufs[i], slow_hbm.at[pl.ds(dst_row, HROWS)], out_sem
                )
            )
        for cp in ins:
            cp.start()
        for i in range(PER_W):
            ins[i].wait()
            outs[i].start()
        for cp in outs:
            cp.wait()

    return k(frames_2d)


def kernel(frames):
    fast = _fast_copy(frames)
    slow = _sc_gather(frames.reshape(C * T * H, W)).reshape(C, S, H, W)
    return (slow, fast)
